# baseline (speedup 1.0000x reference)
import functools

import jax
import jax.numpy as jnp
from jax import lax
from jax.experimental import pallas as pl
from jax.experimental.pallas import tpu as pltpu
from jax.experimental.pallas import tpu_sc as plsc

_N = 10000
_DEG = 16
_D = 128
_E = _N * _DEG

_NW = 32
_CHUNK = 128
_N_PAD = 10240
_ROWS = _DEG * _N_PAD
_NCHUNKS = _ROWS // _CHUNK
_CH0 = 24
_CH1 = 56
_CHMAX = max(_CH0, _CH1)

_BN = 400
_GRID = _N // _BN


_NB = 4


@functools.lru_cache(maxsize=None)
def _make_sc_gather():
    mesh = plsc.VectorSubcoreMesh(core_axis_name="c", subcore_axis_name="s")
    scratch = [pltpu.VMEM((_CHMAX, _CHUNK), jnp.int32)]
    scratch += [pltpu.VMEM((_CHUNK, _D), jnp.float32) for _ in range(_NB)]
    scratch += [pltpu.SemaphoreType.DMA for _ in range(2 * _NB)]

    @functools.partial(
        pl.kernel,
        mesh=mesh,
        out_type=jax.ShapeDtypeStruct((_ROWS, _D), jnp.float32),
        scratch_types=scratch,
    )
    def gather_k(feat_hbm, idx_hbm, out_hbm, idx_v, *rest):
        bufs = rest[:_NB]
        gsem = rest[_NB:2 * _NB]
        ssem = rest[2 * _NB:3 * _NB]
        c = lax.axis_index("c")
        s = lax.axis_index("s")
        n_c = jnp.where(c == 0, _CH0, _CH1)
        chunk0 = jnp.where(c == 0, s * _CH0, 16 * _CH0 + s * _CH1)
        row0 = chunk0 * _CHUNK

        pltpu.sync_copy(idx_hbm.at[pl.ds(chunk0, _CHMAX)], idx_v)

        def gather_start(j, b):
            pltpu.async_copy(feat_hbm.at[idx_v.at[j]], bufs[b], gsem[b])

        def gather_wait(j, b):
            pltpu.make_async_copy(feat_hbm.at[idx_v.at[j]], bufs[b],
                                  gsem[b]).wait()

        for b in range(_NB):
            gather_start(b, b)

        def steady(g, carry):
            for b in range(_NB):
                j = g * _NB + b
                gather_wait(j, b)
                h = pltpu.async_copy(
                    bufs[b], out_hbm.at[pl.ds(row0 + j * _CHUNK, _CHUNK)],
                    ssem[b])
                h.wait()
                gather_start(j + _NB, b)
            return carry

        n_steady = n_c // _NB - 1
        lax.fori_loop(0, n_steady, steady, 0, unroll=False)

        for b in range(_NB):
            j = n_steady * _NB + b
            gather_wait(j, b)
            pltpu.async_copy(
                bufs[b], out_hbm.at[pl.ds(row0 + j * _CHUNK, _CHUNK)],
                ssem[b]).wait()

    return gather_k


def _gru_body(xs_ref, feat_ref, wih_ref, whh_ref, bih_ref, bhh_ref,
              wself_ref, wneigh_ref, bself_ref, out_ref):
    wih = wih_ref[...]
    whh = whh_ref[...]
    bih = bih_ref[...]
    bhh = bhh_ref[...]
    x_all = xs_ref[...].reshape(_DEG * _BN, _D)
    gi_all = jnp.dot(x_all, wih, preferred_element_type=jnp.float32) + bih
    h = None
    for t in range(_DEG):
        gi = gi_all[t * _BN:(t + 1) * _BN]
        if h is None:
            gh = bhh
        else:
            gh = jnp.dot(h, whh, preferred_element_type=jnp.float32) + bhh
        r = jax.nn.sigmoid(gi[:, :_D] + gh[:, :_D])
        z = jax.nn.sigmoid(gi[:, _D:2 * _D] + gh[:, _D:2 * _D])
        n = jnp.tanh(gi[:, 2 * _D:] + r * gh[:, 2 * _D:])
        if h is None:
            h = (1.0 - z) * n
        else:
            h = (1.0 - z) * n + z * h
    out_ref[...] = (
        jnp.dot(feat_ref[...], wself_ref[...], preferred_element_type=jnp.float32)
        + bself_ref[...]
        + jnp.dot(h, wneigh_ref[...], preferred_element_type=jnp.float32)
    )


def _gru_call(xs3, feat, wih, whh, bih, bhh, wself, wneigh, bself):
    return pl.pallas_call(
        _gru_body,
        grid=(_GRID,),
        in_specs=[
            pl.BlockSpec((_DEG, _BN, _D), lambda i: (0, i, 0)),
            pl.BlockSpec((_BN, _D), lambda i: (i, 0)),
            pl.BlockSpec((_D, 3 * _D), lambda i: (0, 0)),
            pl.BlockSpec((_D, 3 * _D), lambda i: (0, 0)),
            pl.BlockSpec((1, 3 * _D), lambda i: (0, 0)),
            pl.BlockSpec((1, 3 * _D), lambda i: (0, 0)),
            pl.BlockSpec((_D, _D), lambda i: (0, 0)),
            pl.BlockSpec((_D, _D), lambda i: (0, 0)),
            pl.BlockSpec((1, _D), lambda i: (0, 0)),
        ],
        out_specs=pl.BlockSpec((_BN, _D), lambda i: (i, 0)),
        out_shape=jax.ShapeDtypeStruct((_N, _D), jnp.float32),
    )(xs3, feat, wih, whh, bih, bhh, wself, wneigh, bself)


def kernel(feat, edge_index, bn_gamma, bn_beta, bn_mean, bn_var,
           W_ih, W_hh, b_ih, b_hh, W_self, W_neigh):
    scale = bn_gamma * lax.rsqrt(bn_var + 1e-5)
    shift = bn_beta - bn_mean * scale
    wih = scale[:, None] * W_ih.T
    bih = (shift @ W_ih.T + b_ih)[None, :]
    whh = W_hh.T
    bhh = b_hh[None, :]
    wself = scale[:, None] * W_self.T
    bself = (shift @ W_self.T)[None, :]
    wneigh = W_neigh.T

    src = edge_index[0]
    idx = jnp.pad(src.reshape(_N, _DEG).T, ((0, 0), (0, _N_PAD - _N)))
    idx = idx.reshape(_NCHUNKS, _CHUNK)
    idx = jnp.pad(idx, ((0, _CHMAX), (0, 0)))

    xs = _make_sc_gather()(feat, idx)
    xs3 = xs.reshape(_DEG, _N_PAD, _D)
    return _gru_call(xs3, feat, wih, whh, bih, bhh, wself, wneigh, bself)


# speedup vs baseline: 2.0716x; 2.0716x over previous
import functools

import jax
import jax.numpy as jnp
from jax import lax
from jax.experimental import pallas as pl
from jax.experimental.pallas import tpu as pltpu
from jax.experimental.pallas import tpu_sc as plsc

_N = 10000
_DEG = 16
_D = 128
_E = _N * _DEG

_NW = 32
_CHUNK = 128
_N_PAD = 10240
_ROWS = _DEG * _N_PAD
_NCHUNKS = _ROWS // _CHUNK
_CH0 = 40
_CH1 = 40
_CHMAX = max(_CH0, _CH1)

_BN = 400
_GRID = _N // _BN


_NB = 6


@functools.lru_cache(maxsize=None)
def _make_sc_gather():
    mesh = plsc.VectorSubcoreMesh(core_axis_name="c", subcore_axis_name="s")
    scratch = [pltpu.VMEM((_CHMAX, _CHUNK), jnp.int32)]
    scratch += [pltpu.VMEM((_CHUNK, _D), jnp.float32) for _ in range(_NB)]
    scratch += [pltpu.SemaphoreType.DMA for _ in range(2 * _NB)]

    @functools.partial(
        pl.kernel,
        mesh=mesh,
        out_type=jax.ShapeDtypeStruct((_ROWS, _D), jnp.float32),
        scratch_types=scratch,
    )
    def gather_k(feat_hbm, idx_hbm, out_hbm, idx_v, *rest):
        bufs = rest[:_NB]
        gsem = rest[_NB:2 * _NB]
        ssem = rest[2 * _NB:3 * _NB]
        c = lax.axis_index("c")
        s = lax.axis_index("s")
        n_c = jnp.where(c == 0, _CH0, _CH1)
        chunk0 = jnp.where(c == 0, s * _CH0, 16 * _CH0 + s * _CH1)
        row0 = chunk0 * _CHUNK

        pltpu.sync_copy(idx_hbm.at[pl.ds(chunk0, _CHMAX)], idx_v)

        def gather_start(j, b):
            pltpu.async_copy(feat_hbm.at[idx_v.at[j]], bufs[b], gsem[b])

        def gather_wait(j, b):
            pltpu.make_async_copy(feat_hbm.at[idx_v.at[j]], bufs[b],
                                  gsem[b]).wait()

        for b in range(_NB):
            gather_start(b, b)

        def steady(g, carry):
            for b in range(_NB):
                j = g * _NB + b
                gather_wait(j, b)
                h = pltpu.async_copy(
                    bufs[b], out_hbm.at[pl.ds(row0 + j * _CHUNK, _CHUNK)],
                    ssem[b])
                h.wait()
                gather_start(j + _NB, b)
            return carry

        n_steady = n_c // _NB - 1
        lax.fori_loop(0, n_steady, steady, 0, unroll=False)

        for b in range(_NB):
            j = n_steady * _NB + b
            gather_wait(j, b)
            pltpu.async_copy(
                bufs[b], out_hbm.at[pl.ds(row0 + j * _CHUNK, _CHUNK)],
                ssem[b]).wait()

    return gather_k


def _gru_body(xs_ref, feat_ref, wih_ref, whh_ref, bih_ref, bhh_ref,
              wself_ref, wneigh_ref, bself_ref, out_ref):
    wih = wih_ref[...]
    whh = whh_ref[...]
    bih = bih_ref[...]
    bhh = bhh_ref[...]
    x_all = xs_ref[...].reshape(_DEG * _BN, _D)
    gi_all = jnp.dot(x_all, wih, preferred_element_type=jnp.float32) + bih
    h = None
    for t in range(_DEG):
        gi = gi_all[t * _BN:(t + 1) * _BN]
        if h is None:
            gh = bhh
        else:
            gh = jnp.dot(h, whh, preferred_element_type=jnp.float32) + bhh
        r = jax.nn.sigmoid(gi[:, :_D] + gh[:, :_D])
        z = jax.nn.sigmoid(gi[:, _D:2 * _D] + gh[:, _D:2 * _D])
        n = jnp.tanh(gi[:, 2 * _D:] + r * gh[:, 2 * _D:])
        if h is None:
            h = (1.0 - z) * n
        else:
            h = (1.0 - z) * n + z * h
    out_ref[...] = (
        jnp.dot(feat_ref[...], wself_ref[...], preferred_element_type=jnp.float32)
        + bself_ref[...]
        + jnp.dot(h, wneigh_ref[...], preferred_element_type=jnp.float32)
    )


def _gru_call(xs3, feat, wih, whh, bih, bhh, wself, wneigh, bself):
    return pl.pallas_call(
        _gru_body,
        grid=(_GRID,),
        in_specs=[
            pl.BlockSpec((_DEG, _BN, _D), lambda i: (0, i, 0)),
            pl.BlockSpec((_BN, _D), lambda i: (i, 0)),
            pl.BlockSpec((_D, 3 * _D), lambda i: (0, 0)),
            pl.BlockSpec((_D, 3 * _D), lambda i: (0, 0)),
            pl.BlockSpec((1, 3 * _D), lambda i: (0, 0)),
            pl.BlockSpec((1, 3 * _D), lambda i: (0, 0)),
            pl.BlockSpec((_D, _D), lambda i: (0, 0)),
            pl.BlockSpec((_D, _D), lambda i: (0, 0)),
            pl.BlockSpec((1, _D), lambda i: (0, 0)),
        ],
        out_specs=pl.BlockSpec((_BN, _D), lambda i: (i, 0)),
        out_shape=jax.ShapeDtypeStruct((_N, _D), jnp.float32),
    )(xs3, feat, wih, whh, bih, bhh, wself, wneigh, bself)


def kernel(feat, edge_index, bn_gamma, bn_beta, bn_mean, bn_var,
           W_ih, W_hh, b_ih, b_hh, W_self, W_neigh):
    scale = bn_gamma * lax.rsqrt(bn_var + 1e-5)
    shift = bn_beta - bn_mean * scale
    wih = scale[:, None] * W_ih.T
    bih = (shift @ W_ih.T + b_ih)[None, :]
    whh = W_hh.T
    bhh = b_hh[None, :]
    wself = scale[:, None] * W_self.T
    bself = (shift @ W_self.T)[None, :]
    wneigh = W_neigh.T

    src = edge_index[0]
    idx = jnp.pad(src.reshape(_N, _DEG).T, ((0, 0), (0, _N_PAD - _N)))
    idx = idx.reshape(_NCHUNKS, _CHUNK)
    idx = jnp.pad(idx, ((0, _CHMAX), (0, 0)))

    xs = _make_sc_gather()(feat, idx)
    xs3 = xs.reshape(_DEG, _N_PAD, _D)
    return _gru_call(xs3, feat, wih, whh, bih, bhh, wself, wneigh, bself)
